# baseline (speedup 1.0000x reference)
import jax
import jax.numpy as jnp
from jax import lax
from jax.experimental import pallas as pl
from jax.experimental.pallas import tpu as pltpu

E = 8
NE = 10
H = 1024
FF = 1408
T = 2048
TILE_M = 256
M_T = T // TILE_M


def _moe_kernel(x32_ref, xb_ref, gw_ref, sg_ref, wg_ref, wu_ref, wd_ref,
                out_ref, w_ref):
    e = pl.program_id(0)
    m = pl.program_id(1)
    msl = pl.ds(m * TILE_M, TILE_M)

    @pl.when(e == 0)
    def _router():
        xf = x32_ref[...]
        logits = jnp.dot(xf, gw_ref[...].T, preferred_element_type=jnp.float32)
        p = jax.nn.softmax(logits, axis=-1)
        lanes = lax.broadcasted_iota(jnp.int32, (TILE_M, E), 1)
        m1 = jnp.max(p, axis=-1, keepdims=True)
        i1 = jnp.min(jnp.where(p == m1, lanes, E), axis=-1, keepdims=True)
        sel1 = lanes == i1
        p2 = jnp.where(sel1, -jnp.inf, p)
        m2 = jnp.max(p2, axis=-1, keepdims=True)
        i2 = jnp.min(jnp.where(p2 == m2, lanes, E), axis=-1, keepdims=True)
        sel2 = lanes == i2
        wdense = jnp.where(sel1, m1, 0.0) + jnp.where(sel2, m2, 0.0)
        sg = jax.nn.sigmoid(
            jnp.dot(xf, sg_ref[...].T, preferred_element_type=jnp.float32))
        wall = jnp.concatenate([wdense, sg, sg], axis=1)
        w_ref[0:NE, msl] = wall.T

    xblk = xb_ref[msl, :]
    wg = wg_ref[0]
    wu = wu_ref[0]
    wd = wd_ref[0]
    g = lax.dot_general(xblk, wg, (((1,), (1,)), ((), ())),
                        preferred_element_type=jnp.float32)
    u = lax.dot_general(xblk, wu, (((1,), (1,)), ((), ())),
                        preferred_element_type=jnp.float32)
    h = (jax.nn.silu(g) * u).astype(jnp.bfloat16)
    d = lax.dot_general(h, wd, (((1,), (1,)), ((), ())),
                        preferred_element_type=jnp.float32)
    scale = w_ref[pl.ds(e, 1), msl]
    contrib = d * scale.T

    @pl.when(e == 0)
    def _init():
        out_ref[msl, :] = contrib

    @pl.when(e > 0)
    def _accum():
        out_ref[msl, :] += contrib


@jax.jit
def kernel(hidden_states, gate_w, Wg, Wu, Wd, sWg, sWu, sWd, shared_gate_w):
    b, s_len, h = hidden_states.shape
    x32 = hidden_states.reshape(T, H)
    xb = x32.astype(jnp.bfloat16)
    wg_all = jnp.concatenate(
        [Wg, sWg.reshape(2, FF, H)], axis=0).astype(jnp.bfloat16)
    wu_all = jnp.concatenate(
        [Wu, sWu.reshape(2, FF, H)], axis=0).astype(jnp.bfloat16)
    wd_all = jnp.concatenate(
        [Wd, sWd.reshape(H, 2, FF).transpose(1, 0, 2)],
        axis=0).astype(jnp.bfloat16)

    out = pl.pallas_call(
        _moe_kernel,
        grid=(NE, M_T),
        in_specs=[
            pl.BlockSpec((TILE_M, H),
                         lambda e, m: (jnp.where(e == 0, m, M_T - 1), 0)),
            pl.BlockSpec((T, H), lambda e, m: (0, 0)),
            pl.BlockSpec((E, H), lambda e, m: (0, 0)),
            pl.BlockSpec((1, H), lambda e, m: (0, 0)),
            pl.BlockSpec((1, FF, H), lambda e, m: (e, 0, 0)),
            pl.BlockSpec((1, FF, H), lambda e, m: (e, 0, 0)),
            pl.BlockSpec((1, H, FF), lambda e, m: (e, 0, 0)),
        ],
        out_specs=pl.BlockSpec((T, H), lambda e, m: (0, 0)),
        out_shape=jax.ShapeDtypeStruct((T, H), jnp.float32),
        scratch_shapes=[
            pltpu.VMEM((16, T), jnp.float32),
        ],
        compiler_params=pltpu.CompilerParams(
            dimension_semantics=("arbitrary", "arbitrary"),
            vmem_limit_bytes=110 * 1024 * 1024,
        ),
    )(x32, xb, gate_w, shared_gate_w, wg_all, wu_all, wd_all)
    return out.reshape(b, s_len, h)
